# baseline (speedup 1.0000x reference)
import functools

import jax
import jax.numpy as jnp
from jax import lax
from jax.experimental import pallas as pl
from jax.experimental.pallas import tpu as pltpu
from jax.experimental.pallas import tpu_sc as plsc

D_MODEL = 64
DP = 128
L = 16


@functools.cache
def _make_transpose(V: int, D: int):
    info = plsc.get_sparse_core_info()
    NC, NS = info.num_cores, info.num_subcores
    NW = NC * NS
    W = 128
    n_win = V // W
    TAIL = V - n_win * W
    wins_per_worker = (n_win + NW - 1) // NW
    NBUF = 2

    mesh = plsc.VectorSubcoreMesh(core_axis_name="c", subcore_axis_name="s")

    @functools.partial(
        pl.kernel,
        mesh=mesh,
        out_type=jax.ShapeDtypeStruct((V, DP), jnp.float32),
        scratch_types=[
            pltpu.VMEM((NBUF, D, W), jnp.float32),
            pltpu.VMEM((NBUF, W, DP), jnp.float32),
            pltpu.SemaphoreType.DMA((NBUF,)),
            pltpu.SemaphoreType.DMA((NBUF,)),
        ],
        compiler_params=pltpu.CompilerParams(needs_layout_passes=False),
    )
    def transpose_kernel(tt_hbm, tail_hbm, out_hbm, in_v, out_v, sem_i, sem_o):
        wid = lax.axis_index("s") * NC + lax.axis_index("c")

        def win_id(k):
            return k * NW + wid

        def start_load(k, b):
            @pl.when(win_id(k) < n_win)
            def _():
                pltpu.async_copy(
                    tt_hbm.at[:, pl.ds(win_id(k) * W, W)], in_v.at[b], sem_i.at[b]
                )

        def do_transpose(b, ntok):
            for j16 in range(D // L):
                rows = lax.iota(jnp.int32, L) + (j16 * L)

                @pl.loop(0, ntok, unroll=8)
                def _(t):
                    col = jnp.full((L,), 0, jnp.int32) + t
                    vals = plsc.load_gather(in_v.at[b], [rows, col])
                    out_v[b, t, pl.ds(j16 * L, L)] = vals

        for b in range(NBUF):
            start_load(b, b)

        @pl.loop(0, wins_per_worker, step=NBUF)
        def _(g):
            for b in range(NBUF):
                k = g + b

                @pl.when(win_id(k) < n_win)
                def _():
                    pltpu.make_async_copy(
                        tt_hbm.at[:, pl.ds(0, W)], in_v.at[b], sem_i.at[b]
                    ).wait()

                    @pl.when(k >= NBUF)
                    def _():
                        pltpu.make_async_copy(
                            out_v.at[b], out_hbm.at[pl.ds(0, W)], sem_o.at[b]
                        ).wait()

                    do_transpose(b, W)

                    pltpu.async_copy(
                        out_v.at[b],
                        out_hbm.at[pl.ds(win_id(k) * W, W)],
                        sem_o.at[b],
                    )

                    @pl.when(k + NBUF < wins_per_worker)
                    def _():
                        start_load(k + NBUF, b)

        for b in range(NBUF):
            last_k = wins_per_worker - NBUF + b

            @pl.when(win_id(last_k) < n_win)
            def _():
                pltpu.make_async_copy(
                    out_v.at[b], out_hbm.at[pl.ds(0, W)], sem_o.at[b]
                ).wait()

        if TAIL:
            @pl.when(wid == 0)
            def _():
                pltpu.async_copy(
                    tail_hbm, out_v.at[0, pl.ds(0, TAIL)], sem_i.at[0]
                ).wait()
                pltpu.async_copy(
                    out_v.at[0, pl.ds(0, TAIL)],
                    out_hbm.at[pl.ds(n_win * W, TAIL)],
                    sem_o.at[0],
                ).wait()

    return transpose_kernel


@functools.cache
def _make_gather(B: int, V: int):
    info = plsc.get_sparse_core_info()
    NC, NS = info.num_cores, info.num_subcores
    NW = NC * NS
    assert B % NW == 0
    b_per_w = B // NW
    C = 200
    NBUF = 4
    LAG = 2
    assert b_per_w % C == 0
    n_chunks = b_per_w // C
    assert n_chunks % NBUF == 0 and n_chunks > NBUF

    mesh = plsc.VectorSubcoreMesh(core_axis_name="c", subcore_axis_name="s")

    @functools.partial(
        pl.kernel,
        mesh=mesh,
        out_type=jax.ShapeDtypeStruct((B, DP), jnp.float32),
        scratch_types=[
            pltpu.VMEM((b_per_w,), jnp.int32),
            pltpu.VMEM((NBUF, C, DP), jnp.float32),
            pltpu.SemaphoreType.DMA,
            pltpu.SemaphoreType.DMA((NBUF,)),
            pltpu.SemaphoreType.DMA((NBUF,)),
        ],
    )
    def gather_kernel(idx_hbm, table_hbm, out_hbm, idx_v, rows_v, sem_i, sem_g, sem_o):
        wid = lax.axis_index("s") * NC + lax.axis_index("c")
        base = wid * b_per_w

        pltpu.async_copy(idx_hbm.at[pl.ds(base, b_per_w)], idx_v, sem_i).wait()

        def start_gather(i, b):
            pltpu.async_copy(
                table_hbm.at[idx_v.at[pl.ds(i * C, C)]], rows_v.at[b], sem_g.at[b]
            )

        def finish_gather_start_writeback(i, b):
            pltpu.make_async_copy(
                table_hbm.at[idx_v.at[pl.ds(i * C, C)]], rows_v.at[b], sem_g.at[b]
            ).wait()
            pltpu.async_copy(
                rows_v.at[b], out_hbm.at[pl.ds(base + i * C, C)], sem_o.at[b]
            )

        @pl.loop(0, n_chunks, step=NBUF)
        def _(g):
            for b in range(NBUF):
                i = g + b

                @pl.when(i >= NBUF)
                def _():
                    pltpu.make_async_copy(
                        rows_v.at[b], out_hbm.at[pl.ds(base, C)], sem_o.at[b]
                    ).wait()

                start_gather(i, b)

                @pl.when(i >= LAG)
                def _():
                    finish_gather_start_writeback(i - LAG, (b - LAG) % NBUF)

        for j in range(LAG):
            i = n_chunks - LAG + j
            finish_gather_start_writeback(i, i % NBUF)
        for b in range(NBUF):
            pltpu.make_async_copy(
                rows_v.at[b], out_hbm.at[pl.ds(base, C)], sem_o.at[b]
            ).wait()

    return gather_kernel


def kernel(x, table):
    B = x.shape[0] * x.shape[1]
    V = table.shape[0]
    n_tail = V % 128
    tail = jnp.pad(table[V - n_tail:], ((0, 0), (0, DP - D_MODEL)))
    table_p = _make_transpose(V, D_MODEL)(table.T, tail)
    out = _make_gather(B, V)(x.reshape(B), table_p)
    return out.reshape(x.shape[0], x.shape[1], DP)[:, :, :D_MODEL]


# speedup vs baseline: 2.1250x; 2.1250x over previous
import functools

import jax
import jax.numpy as jnp
from jax import lax
from jax.experimental import pallas as pl
from jax.experimental.pallas import tpu as pltpu
from jax.experimental.pallas import tpu_sc as plsc

D_MODEL = 64


@functools.cache
def _make_gather(B: int, S: int, V: int, D: int):
    info = plsc.get_sparse_core_info()
    NC, NS = info.num_cores, info.num_subcores
    NW = NC * NS
    assert B % NW == 0
    rows_per_w = B // NW
    NBUF = 4
    LAG = 2
    assert rows_per_w % NBUF == 0 and rows_per_w > NBUF

    mesh = plsc.VectorSubcoreMesh(core_axis_name="c", subcore_axis_name="s")

    @functools.partial(
        pl.kernel,
        mesh=mesh,
        out_type=jax.ShapeDtypeStruct((B, S, 2 * D), jnp.float32),
        scratch_types=[
            pltpu.VMEM((rows_per_w, S), jnp.int32),
            pltpu.VMEM((NBUF, S, D), jnp.float32),
            pltpu.SemaphoreType.DMA,
            pltpu.SemaphoreType.DMA((NBUF,)),
            pltpu.SemaphoreType.DMA((NBUF,)),
        ],
        compiler_params=pltpu.CompilerParams(use_tc_tiling_on_sc=False),
    )
    def gather_kernel(x_hbm, table_hbm, out_hbm, idx_v, rows_v, sem_i, sem_g, sem_o):
        wid = lax.axis_index("s") * NC + lax.axis_index("c")
        base = wid * rows_per_w

        pltpu.async_copy(x_hbm.at[pl.ds(base, rows_per_w)], idx_v, sem_i).wait()

        def start_gather(i, b):
            pltpu.async_copy(
                table_hbm.at[idx_v.at[i]], rows_v.at[b], sem_g.at[b]
            )

        def finish_gather_start_writeback(i, b):
            pltpu.make_async_copy(
                table_hbm.at[idx_v.at[i]], rows_v.at[b], sem_g.at[b]
            ).wait()
            pltpu.async_copy(
                rows_v.at[b], out_hbm.at[base + i, :, pl.ds(0, D)], sem_o.at[b]
            )

        @pl.loop(0, rows_per_w, step=NBUF)
        def _(g):
            for b in range(NBUF):
                i = g + b

                @pl.when(i >= NBUF)
                def _():
                    pltpu.make_async_copy(
                        rows_v.at[b], out_hbm.at[base, :, pl.ds(0, D)], sem_o.at[b]
                    ).wait()

                start_gather(i, b)

                @pl.when(i >= LAG)
                def _():
                    finish_gather_start_writeback(i - LAG, (b - LAG) % NBUF)

        for j in range(LAG):
            i = rows_per_w - LAG + j
            finish_gather_start_writeback(i, i % NBUF)
        for b in range(NBUF):
            pltpu.make_async_copy(
                rows_v.at[b], out_hbm.at[base, :, pl.ds(0, D)], sem_o.at[b]
            ).wait()

    return gather_kernel


def kernel(x, table):
    B, S = x.shape
    out = _make_gather(B, S, table.shape[0], D_MODEL)(x, table)
    return out[:, :, :D_MODEL]
